# Initial kernel scaffold; baseline (speedup 1.0000x reference)
import functools

import jax
import jax.numpy as jnp
from jax import lax
from jax.experimental import pallas as pl
from jax.experimental.pallas import tpu as pltpu
from jax.experimental.pallas import tpu_sc as plsc

_N = 10000
_E = 320000
_D = 128
_DE = 16
_NC = 2
_NS = 16
_NW = _NC * _NS
_EPW = _E // _NW
_B = 80
_NB = _EPW // _B
_RPT = _N // _NS

_f32 = jnp.float32


def _pre_body(z_ref, cpad_ref, ws_ref, wd_ref, e1p_ref, a_ref, b_ref, p1_ref):
    z = z_ref[...]
    a_ref[...] = jnp.dot(z, ws_ref[...], preferred_element_type=_f32)
    b_ref[...] = jnp.dot(z, wd_ref[...], preferred_element_type=_f32)
    p1_ref[...] = jnp.dot(cpad_ref[...], e1p_ref[...], preferred_element_type=_f32)


def _precompute(z, cpad, ws, wd, e1p):
    return pl.pallas_call(
        _pre_body,
        out_shape=(
            jax.ShapeDtypeStruct((_N, _D), _f32),
            jax.ShapeDtypeStruct((_N, _D), _f32),
            jax.ShapeDtypeStruct((_N, _DE), _f32),
        ),
    )(z, cpad, ws, wd, e1p)


def _sc_diff(p1, src, dst):
    mesh = plsc.VectorSubcoreMesh(core_axis_name="c", subcore_axis_name="s")

    @functools.partial(
        pl.kernel,
        out_type=jax.ShapeDtypeStruct((_E, _DE), _f32),
        mesh=mesh,
        scratch_types=[
            pltpu.VMEM((_B,), jnp.int32),
            pltpu.VMEM((_B,), jnp.int32),
            pltpu.VMEM((_B, _DE), _f32),
            pltpu.VMEM((_B, _DE), _f32),
            pltpu.SemaphoreType.DMA,
            pltpu.SemaphoreType.DMA,
        ],
    )
    def k(p1_hbm, src_hbm, dst_hbm, out_hbm, is_v, id_v, ps_v, pd_v, sem0, sem1):
        wid = lax.axis_index("c") * _NS + lax.axis_index("s")

        def body(j, carry):
            base = wid * _EPW + j * _B
            pltpu.sync_copy(src_hbm.at[pl.ds(base, _B)], is_v)
            pltpu.sync_copy(dst_hbm.at[pl.ds(base, _B)], id_v)
            ca = pltpu.async_copy(p1_hbm.at[is_v], ps_v, sem0)
            cb = pltpu.async_copy(p1_hbm.at[id_v], pd_v, sem1)
            ca.wait()
            cb.wait()

            def inner(i, c2):
                sl = pl.ds(0, _DE)
                pd_v[i, sl] = pd_v[i, sl] - ps_v[i, sl]
                return c2

            lax.fori_loop(0, _B, inner, 0)
            pltpu.sync_copy(pd_v, out_hbm.at[pl.ds(base, _B)])
            return carry

        lax.fori_loop(0, _NB, body, 0)

    return k(p1, src, dst)


_RB = 8000


def _emlp_body(d1_ref, eb1_ref, ew2_ref, eb2_ref, we_ref, mb1_ref, c_ref):
    e1 = jnp.maximum(d1_ref[...] + eb1_ref[...], 0.0)
    e = jnp.maximum(
        jnp.dot(e1, ew2_ref[...], preferred_element_type=_f32) + eb2_ref[...], 0.0
    )
    c_ref[...] = jnp.dot(e, we_ref[...], preferred_element_type=_f32) + mb1_ref[...]


def _emlp(d1, eb1, ew2, eb2, we, mb1):
    nblk = _E // _RB
    return pl.pallas_call(
        _emlp_body,
        grid=(nblk,),
        in_specs=[
            pl.BlockSpec((_RB, _DE), lambda i: (i, 0)),
            pl.BlockSpec((1, _DE), lambda i: (0, 0)),
            pl.BlockSpec((_DE, _DE), lambda i: (0, 0)),
            pl.BlockSpec((1, _DE), lambda i: (0, 0)),
            pl.BlockSpec((_DE, _D), lambda i: (0, 0)),
            pl.BlockSpec((1, _D), lambda i: (0, 0)),
        ],
        out_specs=pl.BlockSpec((_RB, _D), lambda i: (i, 0)),
        out_shape=jax.ShapeDtypeStruct((_E, _D), _f32),
    )(d1, eb1, ew2, eb2, we, mb1)


def _sc_main(a, b, c, src, dst, zs, zc):
    mesh = plsc.VectorSubcoreMesh(core_axis_name="c", subcore_axis_name="s")

    @functools.partial(
        pl.kernel,
        out_type=(
            jax.ShapeDtypeStruct((_NC, _N, _D), _f32),
            jax.ShapeDtypeStruct((_NC, _N, _DE), _f32),
        ),
        mesh=mesh,
        scratch_types=[
            pltpu.VMEM((_B,), jnp.int32),
            pltpu.VMEM((_B,), jnp.int32),
            pltpu.VMEM((_B, _D), _f32),
            pltpu.VMEM((_B, _D), _f32),
            pltpu.VMEM((_B, _D), _f32),
            pltpu.VMEM((_B, _DE), _f32),
            pltpu.VMEM((125, _D), _f32),
            pltpu.VMEM((_RPT, _DE), _f32),
            pltpu.VMEM_SHARED((_N, _D), _f32),
            pltpu.VMEM_SHARED((_N, _DE), _f32),
            pltpu.SemaphoreType.DMA,
            pltpu.SemaphoreType.DMA,
            pltpu.SemaphoreType.DMA,
        ],
    )
    def k(a_hbm, b_hbm, c_hbm, src_hbm, dst_hbm, zs_hbm, zc_hbm,
          s_out, cnt_out,
          is_v, id_v, ra, rb, rc, ones_v, obuf, cbuf, s_sh, c_sh,
          sem_a, sem_b, sem_c):
        cid = lax.axis_index("c")
        sid = lax.axis_index("s")
        wid = cid * _NS + sid

        @pl.when(sid == 0)
        def _():
            pltpu.sync_copy(zs_hbm, s_sh)
            pltpu.sync_copy(zc_hbm, c_sh)

        def fill(i, carry):
            ones_v[i, pl.ds(0, _DE)] = jnp.full((_DE,), 1.0, _f32)
            return carry

        lax.fori_loop(0, _B, fill, 0)
        plsc.subcore_barrier()

        def body(j, carry):
            base = wid * _EPW + j * _B
            pltpu.sync_copy(src_hbm.at[pl.ds(base, _B)], is_v)
            pltpu.sync_copy(dst_hbm.at[pl.ds(base, _B)], id_v)
            ca = pltpu.async_copy(a_hbm.at[is_v], ra, sem_a)
            cb = pltpu.async_copy(b_hbm.at[id_v], rb, sem_b)
            cc = pltpu.async_copy(c_hbm.at[pl.ds(base, _B)], rc, sem_c)
            ca.wait()
            cb.wait()
            cc.wait()

            def inner(i, c2):
                for t in range(_D // 16):
                    sl = pl.ds(t * 16, 16)
                    ra[i, sl] = jnp.maximum(ra[i, sl] + rb[i, sl] + rc[i, sl], 0.0)
                return c2

            lax.fori_loop(0, _B, inner, 0)
            pltpu.sync_copy(ra, s_sh.at[id_v], add=True)
            pltpu.sync_copy(ones_v, c_sh.at[id_v], add=True)
            return carry

        lax.fori_loop(0, _NB, body, 0)
        plsc.subcore_barrier()

        r0 = sid * _RPT
        for kk in range(5):
            pltpu.sync_copy(s_sh.at[pl.ds(r0 + kk * 125, 125)], obuf)
            pltpu.sync_copy(obuf, s_out.at[cid, pl.ds(r0 + kk * 125, 125)])
        pltpu.sync_copy(c_sh.at[pl.ds(r0, _RPT)], cbuf)
        pltpu.sync_copy(cbuf, cnt_out.at[cid, pl.ds(r0, _RPT)])

    return k(a, b, c, src, dst, zs, zc)


_FB = 2000


def _final_body(sp_ref, cp_ref, z_ref, mw2_ref, mb2_ref, uz_ref, um_ref,
                ub1_ref, o_ref):
    s = sp_ref[0] + sp_ref[1]
    cnt = cp_ref[0, :, 0:1] + cp_ref[1, :, 0:1]
    s2 = jnp.dot(s, mw2_ref[...], preferred_element_type=_f32) + cnt * mb2_ref[...]
    m = s2 / jnp.maximum(cnt, 1.0)
    o_ref[...] = jnp.maximum(
        jnp.dot(z_ref[...], uz_ref[...], preferred_element_type=_f32)
        + jnp.dot(m, um_ref[...], preferred_element_type=_f32)
        + ub1_ref[...],
        0.0,
    )


def _final(sp, cp, z, mw2, mb2, uz, um, ub1):
    nblk = _N // _FB
    return pl.pallas_call(
        _final_body,
        grid=(nblk,),
        in_specs=[
            pl.BlockSpec((_NC, _FB, _D), lambda i: (0, i, 0)),
            pl.BlockSpec((_NC, _FB, _DE), lambda i: (0, i, 0)),
            pl.BlockSpec((_FB, _D), lambda i: (i, 0)),
            pl.BlockSpec((_D, _D), lambda i: (0, 0)),
            pl.BlockSpec((1, _D), lambda i: (0, 0)),
            pl.BlockSpec((_D, _D), lambda i: (0, 0)),
            pl.BlockSpec((_D, _D), lambda i: (0, 0)),
            pl.BlockSpec((1, _D), lambda i: (0, 0)),
        ],
        out_specs=pl.BlockSpec((_FB, _D), lambda i: (i, 0)),
        out_shape=jax.ShapeDtypeStruct((_N, _D), _f32),
    )(sp, cp, z, mw2, mb2, uz, um, ub1)


def kernel(z, centroids, edge_index, e_w1, e_b1, e_w2, e_b2,
           m_w1, m_b1, m_w2, m_b2, u_w1, u_b1):
    src = edge_index[0]
    dst = edge_index[1]
    ws = m_w1[:_D]
    wd = m_w1[_D:2 * _D]
    we = m_w1[2 * _D:]
    cpad = jnp.concatenate([centroids, jnp.zeros((_N, 5), _f32)], axis=1)
    e1p = jnp.concatenate([e_w1, jnp.zeros((5, _DE), _f32)], axis=0)

    a, b, p1 = _precompute(z, cpad, ws, wd, e1p)
    d1 = _sc_diff(p1, src, dst)
    c = _emlp(d1, e_b1.reshape(1, -1), e_w2, e_b2.reshape(1, -1), we,
              m_b1.reshape(1, -1))
    zs = jnp.zeros((_N, _D), _f32)
    zc = jnp.zeros((_N, _DE), _f32)
    sp, cp = _sc_main(a, b, c, src, dst, zs, zc)
    out = _final(sp, cp, z, m_w2, m_b2.reshape(1, -1), u_w1[:_D], u_w1[_D:],
                 u_b1.reshape(1, -1))
    return out


# speedup vs baseline: 3.9962x; 3.9962x over previous
import functools

import jax
import jax.numpy as jnp
from jax import lax
from jax.experimental import pallas as pl
from jax.experimental.pallas import tpu as pltpu
from jax.experimental.pallas import tpu_sc as plsc

_N = 10000
_E = 320000
_D = 128
_DE = 16
_NC = 2
_NS = 16
_NW = _NC * _NS
_EPW = _E // _NW
_B = 80
_NB = _EPW // _B
_RPT = _N // _NS

_f32 = jnp.float32


def _pre_body(z_ref, cpad_ref, ws_ref, wd_ref, e1p_ref, a_ref, b_ref, p1_ref):
    z = z_ref[...]
    a_ref[...] = jnp.dot(z, ws_ref[...], preferred_element_type=_f32)
    b_ref[...] = jnp.dot(z, wd_ref[...], preferred_element_type=_f32)
    p1_ref[...] = jnp.dot(cpad_ref[...], e1p_ref[...], preferred_element_type=_f32)


def _precompute(z, cpad, ws, wd, e1p):
    return pl.pallas_call(
        _pre_body,
        out_shape=(
            jax.ShapeDtypeStruct((_N, _D), _f32),
            jax.ShapeDtypeStruct((_N, _D), _f32),
            jax.ShapeDtypeStruct((_N, _DE), _f32),
        ),
    )(z, cpad, ws, wd, e1p)


def _sc_diff(p1, src, dst):
    mesh = plsc.VectorSubcoreMesh(core_axis_name="c", subcore_axis_name="s")

    @functools.partial(
        pl.kernel,
        out_type=jax.ShapeDtypeStruct((_E, _DE), _f32),
        mesh=mesh,
        compiler_params=pltpu.CompilerParams(use_tc_tiling_on_sc=False),
        scratch_types=[
            pltpu.VMEM((_B,), jnp.int32),
            pltpu.VMEM((_B,), jnp.int32),
            pltpu.VMEM((_B, _DE), _f32),
            pltpu.VMEM((_B, _DE), _f32),
            pltpu.SemaphoreType.DMA,
            pltpu.SemaphoreType.DMA,
        ],
    )
    def k(p1_hbm, src_hbm, dst_hbm, out_hbm, is_v, id_v, ps_v, pd_v, sem0, sem1):
        wid = lax.axis_index("c") * _NS + lax.axis_index("s")

        def body(j, carry):
            base = wid * _EPW + j * _B
            pltpu.sync_copy(src_hbm.at[pl.ds(base, _B)], is_v)
            pltpu.sync_copy(dst_hbm.at[pl.ds(base, _B)], id_v)
            ca = pltpu.async_copy(p1_hbm.at[is_v], ps_v, sem0)
            cb = pltpu.async_copy(p1_hbm.at[id_v], pd_v, sem1)
            ca.wait()
            cb.wait()

            def inner(i, c2):
                sl = pl.ds(0, _DE)
                pd_v[i, sl] = pd_v[i, sl] - ps_v[i, sl]
                return c2

            lax.fori_loop(0, _B, inner, 0)
            pltpu.sync_copy(pd_v, out_hbm.at[pl.ds(base, _B)])
            return carry

        lax.fori_loop(0, _NB, body, 0)

    return k(p1, src, dst)


_RB = 8000


def _emlp_body(d1_ref, eb1_ref, ew2_ref, eb2_ref, we_ref, mb1_ref, c_ref):
    e1 = jnp.maximum(d1_ref[...] + eb1_ref[...], 0.0)
    e = jnp.maximum(
        jnp.dot(e1, ew2_ref[...], preferred_element_type=_f32) + eb2_ref[...], 0.0
    )
    c_ref[...] = jnp.dot(e, we_ref[...], preferred_element_type=_f32) + mb1_ref[...]


def _emlp(d1, eb1, ew2, eb2, we, mb1):
    nblk = _E // _RB
    return pl.pallas_call(
        _emlp_body,
        grid=(nblk,),
        in_specs=[
            pl.BlockSpec((_RB, _DE), lambda i: (i, 0)),
            pl.BlockSpec((1, _DE), lambda i: (0, 0)),
            pl.BlockSpec((_DE, _DE), lambda i: (0, 0)),
            pl.BlockSpec((1, _DE), lambda i: (0, 0)),
            pl.BlockSpec((_DE, _D), lambda i: (0, 0)),
            pl.BlockSpec((1, _D), lambda i: (0, 0)),
        ],
        out_specs=pl.BlockSpec((_RB, _D), lambda i: (i, 0)),
        out_shape=jax.ShapeDtypeStruct((_E, _D), _f32),
    )(d1, eb1, ew2, eb2, we, mb1)


def _sc_main(a, b, c, src, dst):
    mesh = plsc.VectorSubcoreMesh(core_axis_name="c", subcore_axis_name="s")

    @functools.partial(
        pl.kernel,
        out_type=(
            jax.ShapeDtypeStruct((_NC, _N, _D), _f32),
            jax.ShapeDtypeStruct((_NC, _N, _DE), _f32),
        ),
        mesh=mesh,
        compiler_params=pltpu.CompilerParams(use_tc_tiling_on_sc=False),
        scratch_types=[
            pltpu.VMEM((_B,), jnp.int32),
            pltpu.VMEM((_B,), jnp.int32),
            pltpu.VMEM((_B, _D), _f32),
            pltpu.VMEM((_B, _D), _f32),
            pltpu.VMEM((_B, _D), _f32),
            pltpu.VMEM((_B, _DE), _f32),
            pltpu.VMEM_SHARED((_N, _D), _f32),
            pltpu.VMEM_SHARED((_N, _DE), _f32),
            pltpu.SemaphoreType.DMA,
            pltpu.SemaphoreType.DMA,
            pltpu.SemaphoreType.DMA,
        ],
    )
    def k(a_hbm, b_hbm, c_hbm, src_hbm, dst_hbm,
          s_out, cnt_out,
          is_v, id_v, ra, rb, rc, ones_v, s_sh, c_sh,
          sem_a, sem_b, sem_c):
        cid = lax.axis_index("c")
        sid = lax.axis_index("s")
        wid = cid * _NS + sid
        r0 = sid * _RPT

        def zfill(i, carry):
            for t in range(_D // 16):
                ra[i, pl.ds(t * 16, 16)] = jnp.zeros((16,), _f32)
            ones_v[i, pl.ds(0, _DE)] = jnp.zeros((_DE,), _f32)
            return carry

        lax.fori_loop(0, _B, zfill, 0)
        for kk in range(7):
            pltpu.sync_copy(ra, s_sh.at[pl.ds(r0 + kk * _B, _B)])
            pltpu.sync_copy(ones_v, c_sh.at[pl.ds(r0 + kk * _B, _B)])
        pltpu.sync_copy(ra.at[pl.ds(0, 65)], s_sh.at[pl.ds(r0 + 560, 65)])
        pltpu.sync_copy(ones_v.at[pl.ds(0, 65)], c_sh.at[pl.ds(r0 + 560, 65)])

        def fill(i, carry):
            ones_v[i, pl.ds(0, _DE)] = jnp.full((_DE,), 1.0, _f32)
            return carry

        lax.fori_loop(0, _B, fill, 0)
        plsc.subcore_barrier()

        def body(j, carry):
            base = wid * _EPW + j * _B
            pltpu.sync_copy(src_hbm.at[pl.ds(base, _B)], is_v)
            pltpu.sync_copy(dst_hbm.at[pl.ds(base, _B)], id_v)
            ca = pltpu.async_copy(a_hbm.at[is_v], ra, sem_a)
            cb = pltpu.async_copy(b_hbm.at[id_v], rb, sem_b)
            cc = pltpu.async_copy(c_hbm.at[pl.ds(base, _B)], rc, sem_c)
            ca.wait()
            cb.wait()
            cc.wait()

            def inner(i, c2):
                for t in range(_D // 16):
                    sl = pl.ds(t * 16, 16)
                    ra[i, sl] = jnp.maximum(ra[i, sl] + rb[i, sl] + rc[i, sl], 0.0)
                return c2

            lax.fori_loop(0, _B, inner, 0)
            pltpu.sync_copy(ra, s_sh.at[id_v], add=True)
            pltpu.sync_copy(ones_v, c_sh.at[id_v], add=True)
            return carry

        lax.fori_loop(0, _NB, body, 0)
        plsc.subcore_barrier()

        for kk in range(7):
            pltpu.sync_copy(s_sh.at[pl.ds(r0 + kk * _B, _B)], ra)
            pltpu.sync_copy(ra, s_out.at[cid, pl.ds(r0 + kk * _B, _B)])
            pltpu.sync_copy(c_sh.at[pl.ds(r0 + kk * _B, _B)], ones_v)
            pltpu.sync_copy(ones_v, cnt_out.at[cid, pl.ds(r0 + kk * _B, _B)])
        pltpu.sync_copy(s_sh.at[pl.ds(r0 + 560, 65)], ra.at[pl.ds(0, 65)])
        pltpu.sync_copy(ra.at[pl.ds(0, 65)], s_out.at[cid, pl.ds(r0 + 560, 65)])
        pltpu.sync_copy(c_sh.at[pl.ds(r0 + 560, 65)], ones_v.at[pl.ds(0, 65)])
        pltpu.sync_copy(ones_v.at[pl.ds(0, 65)],
                        cnt_out.at[cid, pl.ds(r0 + 560, 65)])

    return k(a, b, c, src, dst)


_FB = 2000


def _final_body(sp_ref, cp_ref, z_ref, mw2_ref, mb2_ref, uz_ref, um_ref,
                ub1_ref, o_ref):
    s = sp_ref[0] + sp_ref[1]
    cnt = cp_ref[0, :, 0:1] + cp_ref[1, :, 0:1]
    s2 = jnp.dot(s, mw2_ref[...], preferred_element_type=_f32) + cnt * mb2_ref[...]
    m = s2 / jnp.maximum(cnt, 1.0)
    o_ref[...] = jnp.maximum(
        jnp.dot(z_ref[...], uz_ref[...], preferred_element_type=_f32)
        + jnp.dot(m, um_ref[...], preferred_element_type=_f32)
        + ub1_ref[...],
        0.0,
    )


def _final(sp, cp, z, mw2, mb2, uz, um, ub1):
    nblk = _N // _FB
    return pl.pallas_call(
        _final_body,
        grid=(nblk,),
        in_specs=[
            pl.BlockSpec((_NC, _FB, _D), lambda i: (0, i, 0)),
            pl.BlockSpec((_NC, _FB, _DE), lambda i: (0, i, 0)),
            pl.BlockSpec((_FB, _D), lambda i: (i, 0)),
            pl.BlockSpec((_D, _D), lambda i: (0, 0)),
            pl.BlockSpec((1, _D), lambda i: (0, 0)),
            pl.BlockSpec((_D, _D), lambda i: (0, 0)),
            pl.BlockSpec((_D, _D), lambda i: (0, 0)),
            pl.BlockSpec((1, _D), lambda i: (0, 0)),
        ],
        out_specs=pl.BlockSpec((_FB, _D), lambda i: (i, 0)),
        out_shape=jax.ShapeDtypeStruct((_N, _D), _f32),
    )(sp, cp, z, mw2, mb2, uz, um, ub1)


def kernel(z, centroids, edge_index, e_w1, e_b1, e_w2, e_b2,
           m_w1, m_b1, m_w2, m_b2, u_w1, u_b1):
    src = edge_index[0]
    dst = edge_index[1]
    ws = m_w1[:_D]
    wd = m_w1[_D:2 * _D]
    we = m_w1[2 * _D:]
    cpad = jnp.concatenate([centroids, jnp.zeros((_N, 5), _f32)], axis=1)
    e1p = jnp.concatenate([e_w1, jnp.zeros((5, _DE), _f32)], axis=0)

    a, b, p1 = _precompute(z, cpad, ws, wd, e1p)
    d1 = _sc_diff(p1, src, dst)
    c = _emlp(d1, e_b1.reshape(1, -1), e_w2, e_b2.reshape(1, -1), we,
              m_b1.reshape(1, -1))
    sp, cp = _sc_main(a, b, c, src, dst)
    out = _final(sp, cp, z, m_w2, m_b2.reshape(1, -1), u_w1[:_D], u_w1[_D:],
                 u_b1.reshape(1, -1))
    return out


# speedup vs baseline: 5.9445x; 1.4875x over previous
import functools

import jax
import jax.numpy as jnp
from jax import lax
from jax.experimental import pallas as pl
from jax.experimental.pallas import tpu as pltpu
from jax.experimental.pallas import tpu_sc as plsc

_N = 10000
_E = 320000
_D = 128
_DE = 16
_NC = 2
_NS = 16
_NW = _NC * _NS
_EPW = _E // _NW
_B = 40
_NB = _EPW // _B
_RPT = _N // _NS

_f32 = jnp.float32


def _pre_body(z_ref, cpad_ref, ws_ref, wd_ref, e1p_ref, a_ref, b_ref, p1_ref):
    z = z_ref[...]
    a_ref[...] = jnp.dot(z, ws_ref[...], preferred_element_type=_f32)
    b_ref[...] = jnp.dot(z, wd_ref[...], preferred_element_type=_f32)
    p1_ref[...] = jnp.dot(cpad_ref[...], e1p_ref[...], preferred_element_type=_f32)


def _precompute(z, cpad, ws, wd, e1p):
    return pl.pallas_call(
        _pre_body,
        out_shape=(
            jax.ShapeDtypeStruct((_N, _D), _f32),
            jax.ShapeDtypeStruct((_N, _D), _f32),
            jax.ShapeDtypeStruct((_N, _DE), _f32),
        ),
    )(z, cpad, ws, wd, e1p)


def _sc_diff(p1, src, dst):
    mesh = plsc.VectorSubcoreMesh(core_axis_name="c", subcore_axis_name="s")

    @functools.partial(
        pl.kernel,
        out_type=jax.ShapeDtypeStruct((_E, _DE), _f32),
        mesh=mesh,
        compiler_params=pltpu.CompilerParams(use_tc_tiling_on_sc=False),
        scratch_types=(
            [pltpu.VMEM((_B,), jnp.int32)] * 4
            + [pltpu.VMEM((_B, _DE), _f32)] * 4
            + [pltpu.SemaphoreType.DMA] * 10
        ),
    )
    def k(p1_hbm, src_hbm, dst_hbm, out_hbm,
          is0, is1, id0, id1, ps0, ps1, pd0, pd1,
          sia0, sia1, sib0, sib1, sa0, sa1, sb0, sb1, sw0, sw1):
        wid = lax.axis_index("c") * _NS + lax.axis_index("s")
        is_ = (is0, is1)
        id_ = (id0, id1)
        ps = (ps0, ps1)
        pd = (pd0, pd1)
        sia = (sia0, sia1)
        sib = (sib0, sib1)
        sa = (sa0, sa1)
        sb = (sb0, sb1)
        sw = (sw0, sw1)

        def base_of(j):
            return wid * _EPW + j * _B

        def issue_idx(j, p):
            pltpu.async_copy(src_hbm.at[pl.ds(base_of(j), _B)], is_[p], sia[p])
            pltpu.async_copy(dst_hbm.at[pl.ds(base_of(j), _B)], id_[p], sib[p])

        def wait_idx(p):
            pltpu.make_async_copy(src_hbm.at[pl.ds(0, _B)], is_[p], sia[p]).wait()
            pltpu.make_async_copy(dst_hbm.at[pl.ds(0, _B)], id_[p], sib[p]).wait()

        def issue_gather(p):
            pltpu.async_copy(p1_hbm.at[is_[p]], ps[p], sa[p])
            pltpu.async_copy(p1_hbm.at[id_[p]], pd[p], sb[p])

        def wait_gather(p):
            pltpu.make_async_copy(p1_hbm.at[is_[p]], ps[p], sa[p]).wait()
            pltpu.make_async_copy(p1_hbm.at[id_[p]], pd[p], sb[p]).wait()

        issue_idx(0, 0)
        issue_idx(1, 1)
        wait_idx(0)
        issue_gather(0)

        def sub(j, p, k_, first, last):
            q = 1 - p
            if first is None:
                pltpu.make_async_copy(
                    pd[q], out_hbm.at[pl.ds(0, _B)], sw[q]).wait()
            else:
                @pl.when(k_ > 0)
                def _():
                    pltpu.make_async_copy(
                        pd[q], out_hbm.at[pl.ds(0, _B)], sw[q]).wait()
            if last is None:
                wait_idx(q)
                issue_gather(q)
            else:
                @pl.when(k_ < 124)
                def _():
                    wait_idx(q)
                    issue_gather(q)
            wait_gather(p)

            def inner(i, c2):
                sl = pl.ds(0, _DE)
                pd[p][i, sl] = pd[p][i, sl] - ps[p][i, sl]
                return c2

            lax.fori_loop(0, _B, inner, 0)
            pltpu.async_copy(pd[p], out_hbm.at[pl.ds(base_of(j), _B)], sw[p])

            @pl.when(k_ < 124)
            def _():
                issue_idx(j + 2, p)

        def body(k_, carry):
            sub(2 * k_, 0, k_, first=True, last=None)
            sub(2 * k_ + 1, 1, k_, first=None, last=True)
            return carry

        lax.fori_loop(0, _NB // 2, body, 0)
        pltpu.make_async_copy(pd[1], out_hbm.at[pl.ds(0, _B)], sw[1]).wait()

    return k(p1, src, dst)


_RB = 8000


def _emlp_body(d1_ref, eb1_ref, ew2_ref, eb2_ref, we_ref, mb1_ref, c_ref):
    e1 = jnp.maximum(d1_ref[...] + eb1_ref[...], 0.0)
    e = jnp.maximum(
        jnp.dot(e1, ew2_ref[...], preferred_element_type=_f32) + eb2_ref[...], 0.0
    )
    c_ref[...] = jnp.dot(e, we_ref[...], preferred_element_type=_f32) + mb1_ref[...]


def _emlp(d1, eb1, ew2, eb2, we, mb1):
    nblk = _E // _RB
    return pl.pallas_call(
        _emlp_body,
        grid=(nblk,),
        in_specs=[
            pl.BlockSpec((_RB, _DE), lambda i: (i, 0)),
            pl.BlockSpec((1, _DE), lambda i: (0, 0)),
            pl.BlockSpec((_DE, _DE), lambda i: (0, 0)),
            pl.BlockSpec((1, _DE), lambda i: (0, 0)),
            pl.BlockSpec((_DE, _D), lambda i: (0, 0)),
            pl.BlockSpec((1, _D), lambda i: (0, 0)),
        ],
        out_specs=pl.BlockSpec((_RB, _D), lambda i: (i, 0)),
        out_shape=jax.ShapeDtypeStruct((_E, _D), _f32),
    )(d1, eb1, ew2, eb2, we, mb1)


def _sc_main(a, b, c, src, dst):
    mesh = plsc.VectorSubcoreMesh(core_axis_name="c", subcore_axis_name="s")

    @functools.partial(
        pl.kernel,
        out_type=(
            jax.ShapeDtypeStruct((_NC, _N, _D), _f32),
            jax.ShapeDtypeStruct((_NC, _N, _DE), _f32),
        ),
        mesh=mesh,
        compiler_params=pltpu.CompilerParams(use_tc_tiling_on_sc=False),
        scratch_types=(
            [pltpu.VMEM((_B,), jnp.int32)] * 6
            + [pltpu.VMEM((_B, _D), _f32)] * 6
            + [pltpu.VMEM((_B, _DE), _f32)]
            + [pltpu.VMEM_SHARED((_N, _D), _f32)]
            + [pltpu.VMEM_SHARED((_N, _DE), _f32)]
            + [pltpu.SemaphoreType.DMA] * 14
        ),
    )
    def k(a_hbm, b_hbm, c_hbm, src_hbm, dst_hbm,
          s_out, cnt_out,
          is0, is1, id0, id1, sd0, sd1,
          ra0, ra1, rb0, rb1, rc0, rc1, ones_v, s_sh, c_sh,
          sia0, sia1, sib0, sib1, sa0, sa1, sb0, sb1, sc0, sc1,
          ss0, ss1, so0, so1):
        cid = lax.axis_index("c")
        sid = lax.axis_index("s")
        wid = cid * _NS + sid
        r0 = sid * _RPT
        is_ = (is0, is1)
        id_ = (id0, id1)
        sd = (sd0, sd1)
        ra = (ra0, ra1)
        rb = (rb0, rb1)
        rc = (rc0, rc1)
        sia = (sia0, sia1)
        sib = (sib0, sib1)
        sa = (sa0, sa1)
        sb = (sb0, sb1)
        sc = (sc0, sc1)
        ss = (ss0, ss1)
        so = (so0, so1)

        def zfill(i, carry):
            for t in range(_D // 16):
                ra0[i, pl.ds(t * 16, 16)] = jnp.zeros((16,), _f32)
            ones_v[i, pl.ds(0, _DE)] = jnp.zeros((_DE,), _f32)
            return carry

        lax.fori_loop(0, _B, zfill, 0)
        for kk in range(15):
            pltpu.sync_copy(ra0, s_sh.at[pl.ds(r0 + kk * _B, _B)])
            pltpu.sync_copy(ones_v, c_sh.at[pl.ds(r0 + kk * _B, _B)])
        pltpu.sync_copy(ra0.at[pl.ds(0, 25)], s_sh.at[pl.ds(r0 + 600, 25)])
        pltpu.sync_copy(ones_v.at[pl.ds(0, 25)], c_sh.at[pl.ds(r0 + 600, 25)])

        def fill(i, carry):
            ones_v[i, pl.ds(0, _DE)] = jnp.full((_DE,), 1.0, _f32)
            return carry

        lax.fori_loop(0, _B, fill, 0)
        plsc.subcore_barrier()

        def base_of(j):
            return wid * _EPW + j * _B

        def issue_idx(j, p):
            pltpu.async_copy(src_hbm.at[pl.ds(base_of(j), _B)], is_[p], sia[p])
            pltpu.async_copy(dst_hbm.at[pl.ds(base_of(j), _B)], id_[p], sib[p])

        def wait_idx(p):
            pltpu.make_async_copy(src_hbm.at[pl.ds(0, _B)], is_[p], sia[p]).wait()
            pltpu.make_async_copy(dst_hbm.at[pl.ds(0, _B)], id_[p], sib[p]).wait()

        def issue_gather(j, p):
            pltpu.async_copy(a_hbm.at[is_[p]], ra[p], sa[p])
            pltpu.async_copy(b_hbm.at[id_[p]], rb[p], sb[p])
            pltpu.async_copy(c_hbm.at[pl.ds(base_of(j), _B)], rc[p], sc[p])

        def wait_gather(p):
            pltpu.make_async_copy(a_hbm.at[is_[p]], ra[p], sa[p]).wait()
            pltpu.make_async_copy(b_hbm.at[id_[p]], rb[p], sb[p]).wait()
            pltpu.make_async_copy(c_hbm.at[pl.ds(0, _B)], rc[p], sc[p]).wait()

        def drain_scatter(p):
            pltpu.make_async_copy(ra[p], s_sh.at[sd[p]], ss[p]).wait()
            pltpu.make_async_copy(ones_v, c_sh.at[sd[p]], so[p]).wait()

        issue_idx(0, 0)
        issue_idx(1, 1)
        wait_idx(0)
        issue_gather(0, 0)

        def sub(j, p, k_, first, last):
            q = 1 - p
            if first is None:
                drain_scatter(q)
            else:
                @pl.when(k_ > 0)
                def _():
                    drain_scatter(q)
            if last is None:
                wait_idx(q)
                issue_gather(j + 1, q)
            else:
                @pl.when(k_ < _NB // 2 - 1)
                def _():
                    wait_idx(q)
                    issue_gather(j + 1, q)
            wait_gather(p)

            for t, off in ((0, 0), (1, 16), (2, 24)):
                sd[p][pl.ds(off, 16)] = id_[p][pl.ds(off, 16)]

            @pl.when(k_ < _NB // 2 - 1)
            def _():
                issue_idx(j + 2, p)

            def inner(i, c2):
                for t in range(_D // 16):
                    sl = pl.ds(t * 16, 16)
                    ra[p][i, sl] = jnp.maximum(
                        ra[p][i, sl] + rb[p][i, sl] + rc[p][i, sl], 0.0)
                return c2

            lax.fori_loop(0, _B, inner, 0)
            pltpu.async_copy(ra[p], s_sh.at[sd[p]], ss[p], add=True)
            pltpu.async_copy(ones_v, c_sh.at[sd[p]], so[p], add=True)

        def body(k_, carry):
            sub(2 * k_, 0, k_, first=True, last=None)
            sub(2 * k_ + 1, 1, k_, first=None, last=True)
            return carry

        lax.fori_loop(0, _NB // 2, body, 0)
        drain_scatter(1)
        plsc.subcore_barrier()

        for kk in range(15):
            pltpu.sync_copy(s_sh.at[pl.ds(r0 + kk * _B, _B)], ra0)
            pltpu.sync_copy(ra0, s_out.at[cid, pl.ds(r0 + kk * _B, _B)])
            pltpu.sync_copy(c_sh.at[pl.ds(r0 + kk * _B, _B)], ones_v)
            pltpu.sync_copy(ones_v, cnt_out.at[cid, pl.ds(r0 + kk * _B, _B)])
        pltpu.sync_copy(s_sh.at[pl.ds(r0 + 600, 25)], ra0.at[pl.ds(0, 25)])
        pltpu.sync_copy(ra0.at[pl.ds(0, 25)], s_out.at[cid, pl.ds(r0 + 600, 25)])
        pltpu.sync_copy(c_sh.at[pl.ds(r0 + 600, 25)], ones_v.at[pl.ds(0, 25)])
        pltpu.sync_copy(ones_v.at[pl.ds(0, 25)],
                        cnt_out.at[cid, pl.ds(r0 + 600, 25)])

    return k(a, b, c, src, dst)


_FB = 2000


def _final_body(sp_ref, cp_ref, z_ref, mw2_ref, mb2_ref, uz_ref, um_ref,
                ub1_ref, o_ref):
    s = sp_ref[0] + sp_ref[1]
    cnt = cp_ref[0, :, 0:1] + cp_ref[1, :, 0:1]
    s2 = jnp.dot(s, mw2_ref[...], preferred_element_type=_f32) + cnt * mb2_ref[...]
    m = s2 / jnp.maximum(cnt, 1.0)
    o_ref[...] = jnp.maximum(
        jnp.dot(z_ref[...], uz_ref[...], preferred_element_type=_f32)
        + jnp.dot(m, um_ref[...], preferred_element_type=_f32)
        + ub1_ref[...],
        0.0,
    )


def _final(sp, cp, z, mw2, mb2, uz, um, ub1):
    nblk = _N // _FB
    return pl.pallas_call(
        _final_body,
        grid=(nblk,),
        in_specs=[
            pl.BlockSpec((_NC, _FB, _D), lambda i: (0, i, 0)),
            pl.BlockSpec((_NC, _FB, _DE), lambda i: (0, i, 0)),
            pl.BlockSpec((_FB, _D), lambda i: (i, 0)),
            pl.BlockSpec((_D, _D), lambda i: (0, 0)),
            pl.BlockSpec((1, _D), lambda i: (0, 0)),
            pl.BlockSpec((_D, _D), lambda i: (0, 0)),
            pl.BlockSpec((_D, _D), lambda i: (0, 0)),
            pl.BlockSpec((1, _D), lambda i: (0, 0)),
        ],
        out_specs=pl.BlockSpec((_FB, _D), lambda i: (i, 0)),
        out_shape=jax.ShapeDtypeStruct((_N, _D), _f32),
    )(sp, cp, z, mw2, mb2, uz, um, ub1)


def kernel(z, centroids, edge_index, e_w1, e_b1, e_w2, e_b2,
           m_w1, m_b1, m_w2, m_b2, u_w1, u_b1):
    src = edge_index[0]
    dst = edge_index[1]
    ws = m_w1[:_D]
    wd = m_w1[_D:2 * _D]
    we = m_w1[2 * _D:]
    cpad = jnp.concatenate([centroids, jnp.zeros((_N, 5), _f32)], axis=1)
    e1p = jnp.concatenate([e_w1, jnp.zeros((5, _DE), _f32)], axis=0)

    a, b, p1 = _precompute(z, cpad, ws, wd, e1p)
    d1 = _sc_diff(p1, src, dst)
    c = _emlp(d1, e_b1.reshape(1, -1), e_w2, e_b2.reshape(1, -1), we,
              m_b1.reshape(1, -1))
    sp, cp = _sc_main(a, b, c, src, dst)
    out = _final(sp, cp, z, m_w2, m_b2.reshape(1, -1), u_w1[:_D], u_w1[_D:],
                 u_b1.reshape(1, -1))
    return out
